# baseline (speedup 1.0000x reference)
import functools

import jax
import jax.numpy as jnp
from jax import lax
from jax.experimental import pallas as pl
from jax.experimental.pallas import tpu as pltpu
from jax.experimental.pallas import tpu_sc as plsc

_info = plsc.get_sparse_core_info()
_NC, _NS, _L = _info.num_cores, _info.num_subcores, _info.num_lanes
_NW = _NC * _NS


@functools.partial(jax.jit, static_argnums=(2, 3, 4))
def _gather_rows(table_flat, idx_pad, C, BANK, BPW):
    PAD = idx_pad.shape[0]
    D = table_flat.shape[1]
    mesh = plsc.VectorSubcoreMesh(core_axis_name="c", subcore_axis_name="s")

    @functools.partial(
        pl.kernel,
        mesh=mesh,
        out_type=jax.ShapeDtypeStruct((PAD, D), jnp.float32),
        scratch_types=[
            pltpu.VMEM((BPW,), jnp.int32),
            pltpu.VMEM((BPW,), jnp.int32),
            pltpu.VMEM((BPW, D), jnp.float32),
            pltpu.SemaphoreType.DMA,
        ],
    )
    def k(table_hbm, idx_hbm, out_hbm, idx_v, flat_v, rows_v, sem):
        wid = lax.axis_index("s") * _NC + lax.axis_index("c")
        base = wid * BPW
        pltpu.sync_copy(idx_hbm.at[pl.ds(base, BPW)], idx_v)
        for j in range(BPW // _L):
            cat = base + j * _L + lax.iota(jnp.int32, _L)
            cat = jnp.minimum(cat, C - 1)
            flat_v[pl.ds(j * _L, _L)] = cat * BANK + idx_v[pl.ds(j * _L, _L)]
        pltpu.async_copy(table_hbm.at[flat_v], rows_v, sem).wait()
        pltpu.sync_copy(rows_v, out_hbm.at[pl.ds(base, BPW)])

    return k(table_flat, idx_pad)


def kernel(table, indices):
    C, BANK, D = table.shape
    BPW = -(-C // (_NW * _L)) * _L
    PAD = _NW * BPW
    table_flat = table.reshape(C * BANK, D)
    idx_pad = jnp.zeros((PAD,), jnp.int32).at[:C].set(indices.astype(jnp.int32))
    out_pad = _gather_rows(table_flat, idx_pad, C, BANK, BPW)
    return out_pad[:C]
